# baseline (speedup 1.0000x reference)
import functools

import jax
import jax.numpy as jnp
from jax import lax
from jax.experimental import pallas as pl
from jax.experimental.pallas import tpu as pltpu
from jax.experimental.pallas import tpu_sc as plsc

N_ROWS = 8192
DIM = 4096
NC = 2
NS = 16
L = 16
NW = NC * NS
ROWS_PER_W = N_ROWS // NW
RBLK = 8

_mesh = plsc.VectorSubcoreMesh(core_axis_name="c", subcore_axis_name="s")


@functools.partial(
    pl.kernel,
    mesh=_mesh,
    out_type=jax.ShapeDtypeStruct((N_ROWS * DIM,), jnp.float32),
    compiler_params=pltpu.CompilerParams(needs_layout_passes=False),
    scratch_types=[
        pltpu.VMEM((DIM,), jnp.int32),
        pltpu.VMEM((RBLK * DIM,), jnp.float32),
        pltpu.VMEM((RBLK * DIM,), jnp.float32),
    ],
)
def _permute_cols(x_flat, perm_hbm, out_flat, perm_v, in_v, out_v):
    wid = lax.axis_index("s") * NC + lax.axis_index("c")
    base = wid * ROWS_PER_W
    pltpu.sync_copy(perm_hbm, perm_v)

    def block_body(b, carry):
        row0 = base + b * RBLK
        pltpu.sync_copy(x_flat.at[pl.ds(row0 * DIM, RBLK * DIM)], in_v)

        def j_body(j, c):
            col0 = j * L
            idx = perm_v[pl.ds(col0, L)]
            for r in range(RBLK):
                vals = plsc.load_gather(in_v, [idx + r * DIM])
                out_v[pl.ds(r * DIM + col0, L)] = vals
            return c

        lax.fori_loop(0, DIM // L, j_body, 0)
        pltpu.sync_copy(out_v, out_flat.at[pl.ds(row0 * DIM, RBLK * DIM)])
        return carry

    lax.fori_loop(0, ROWS_PER_W // RBLK, block_body, 0)


def kernel(x, perm):
    out = _permute_cols(x.reshape(-1), perm.astype(jnp.int32))
    return out.reshape(N_ROWS, DIM), 0


# speedup vs baseline: 1.2035x; 1.2035x over previous
import functools

import jax
import jax.numpy as jnp
from jax import lax
from jax.experimental import pallas as pl
from jax.experimental.pallas import tpu as pltpu
from jax.experimental.pallas import tpu_sc as plsc

N_ROWS = 8192
DIM = 4096
NC = 2
NS = 16
L = 16
NW = NC * NS
ROWS_PER_W = N_ROWS // NW
RBLK = 4
NB = ROWS_PER_W // RBLK
BLK = RBLK * DIM

_mesh = plsc.VectorSubcoreMesh(core_axis_name="c", subcore_axis_name="s")


@functools.partial(
    pl.kernel,
    mesh=_mesh,
    out_type=jax.ShapeDtypeStruct((N_ROWS * DIM,), jnp.float32),
    compiler_params=pltpu.CompilerParams(needs_layout_passes=False),
    scratch_types=[
        pltpu.VMEM((DIM,), jnp.int32),
        pltpu.VMEM((BLK,), jnp.float32),
        pltpu.VMEM((BLK,), jnp.float32),
        pltpu.VMEM((BLK,), jnp.float32),
        pltpu.VMEM((BLK,), jnp.float32),
        pltpu.SemaphoreType.DMA,
        pltpu.SemaphoreType.DMA,
        pltpu.SemaphoreType.DMA,
        pltpu.SemaphoreType.DMA,
    ],
)
def _permute_cols(x_flat, perm_hbm, out_flat, perm_v,
                  in0, in1, out0, out1,
                  sem_in0, sem_in1, sem_out0, sem_out1):
    wid = lax.axis_index("s") * NC + lax.axis_index("c")
    base = wid * ROWS_PER_W * DIM
    pltpu.sync_copy(perm_hbm, perm_v)

    def src(b):
        return x_flat.at[pl.ds(base + b * BLK, BLK)]

    def dst(b):
        return out_flat.at[pl.ds(base + b * BLK, BLK)]

    def compute(in_v, out_v):
        def j_body(j, c):
            col0 = j * (2 * L)
            idx_a = perm_v[pl.ds(col0, L)]
            idx_b = perm_v[pl.ds(col0 + L, L)]
            for r in range(RBLK):
                out_v[pl.ds(r * DIM + col0, L)] = plsc.load_gather(
                    in_v, [idx_a + r * DIM])
                out_v[pl.ds(r * DIM + col0 + L, L)] = plsc.load_gather(
                    in_v, [idx_b + r * DIM])
            return c
        lax.fori_loop(0, DIM // (2 * L), j_body, 0)

    pltpu.async_copy(src(0), in0, sem_in0)

    def pair_body(p, carry):
        b = 2 * p
        pltpu.async_copy(src(b + 1), in1, sem_in1)
        pltpu.make_async_copy(src(b), in0, sem_in0).wait()

        @pl.when(p > 0)
        def _():
            pltpu.make_async_copy(out0, dst(b - 2), sem_out0).wait()

        compute(in0, out0)
        pltpu.async_copy(out0, dst(b), sem_out0)

        @pl.when(p < NB // 2 - 1)
        def _():
            pltpu.async_copy(src(b + 2), in0, sem_in0)

        pltpu.make_async_copy(src(b + 1), in1, sem_in1).wait()

        @pl.when(p > 0)
        def _():
            pltpu.make_async_copy(out1, dst(b - 1), sem_out1).wait()

        compute(in1, out1)
        pltpu.async_copy(out1, dst(b + 1), sem_out1)
        return carry

    lax.fori_loop(0, NB // 2, pair_body, 0)
    pltpu.make_async_copy(out0, dst(NB - 2), sem_out0).wait()
    pltpu.make_async_copy(out1, dst(NB - 1), sem_out1).wait()


def kernel(x, perm):
    out = _permute_cols(x.reshape(-1), perm.astype(jnp.int32))
    return out.reshape(N_ROWS, DIM), 0


# speedup vs baseline: 1.4719x; 1.2230x over previous
import functools

import jax
import jax.numpy as jnp
from jax import lax
from jax.experimental import pallas as pl
from jax.experimental.pallas import tpu as pltpu
from jax.experimental.pallas import tpu_sc as plsc

N_ROWS = 8192
DIM = 4096
NC = 2
NS = 16
L = 16
NW = NC * NS
ROWS_PER_W = N_ROWS // NW
RBLK = 8
NB = ROWS_PER_W // RBLK

_mesh = plsc.VectorSubcoreMesh(core_axis_name="c", subcore_axis_name="s")


@functools.partial(
    pl.kernel,
    mesh=_mesh,
    out_type=jax.ShapeDtypeStruct((N_ROWS, DIM), jnp.float32),
    compiler_params=pltpu.CompilerParams(needs_layout_passes=False),
    scratch_types=[
        pltpu.VMEM((DIM,), jnp.int32),
        pltpu.VMEM((RBLK, DIM), jnp.float32),
        pltpu.VMEM((RBLK, DIM), jnp.float32),
    ],
)
def _permute_cols(x_hbm, perm_hbm, out_hbm, perm_v, in_v, out_v):
    wid = lax.axis_index("s") * NC + lax.axis_index("c")
    base = wid * ROWS_PER_W
    pltpu.sync_copy(perm_hbm, perm_v)

    def block_body(b, carry):
        row0 = base + b * RBLK
        pltpu.sync_copy(x_hbm.at[pl.ds(row0, RBLK)], in_v)

        def j_body(j, c):
            col0 = j * L
            idx = perm_v[pl.ds(col0, L)]
            for r in range(RBLK):
                row_idx = jnp.full((L,), r, jnp.int32)
                vals = plsc.load_gather(in_v, [row_idx, idx])
                out_v[r, pl.ds(col0, L)] = vals
            return c

        lax.fori_loop(0, DIM // L, j_body, 0)
        pltpu.sync_copy(out_v, out_hbm.at[pl.ds(row0, RBLK)])
        return carry

    lax.fori_loop(0, NB, block_body, 0)


def kernel(x, perm):
    out = _permute_cols(x, perm.astype(jnp.int32))
    return out, 0


# speedup vs baseline: 1.8739x; 1.2731x over previous
import functools

import jax
import jax.numpy as jnp
from jax import lax
from jax.experimental import pallas as pl
from jax.experimental.pallas import tpu as pltpu
from jax.experimental.pallas import tpu_sc as plsc

N_ROWS = 8192
DIM = 4096
NC = 2
NS = 16
L = 16
NW = NC * NS
ROWS_PER_W = N_ROWS // NW
RBLK = 4
NB = ROWS_PER_W // RBLK

_mesh = plsc.VectorSubcoreMesh(core_axis_name="c", subcore_axis_name="s")


@functools.partial(
    pl.kernel,
    mesh=_mesh,
    out_type=jax.ShapeDtypeStruct((N_ROWS, DIM), jnp.float32),
    compiler_params=pltpu.CompilerParams(needs_layout_passes=False),
    scratch_types=[
        pltpu.VMEM((DIM,), jnp.int32),
        pltpu.VMEM((RBLK, DIM), jnp.float32),
        pltpu.VMEM((RBLK, DIM), jnp.float32),
        pltpu.VMEM((RBLK, DIM), jnp.float32),
        pltpu.VMEM((RBLK, DIM), jnp.float32),
        pltpu.SemaphoreType.DMA,
        pltpu.SemaphoreType.DMA,
        pltpu.SemaphoreType.DMA,
        pltpu.SemaphoreType.DMA,
    ],
)
def _permute_cols(x_hbm, perm_hbm, out_hbm, perm_v,
                  in0, in1, out0, out1,
                  sem_in0, sem_in1, sem_out0, sem_out1):
    wid = lax.axis_index("s") * NC + lax.axis_index("c")
    base = wid * ROWS_PER_W
    pltpu.sync_copy(perm_hbm, perm_v)

    def src(b):
        return x_hbm.at[pl.ds(base + b * RBLK, RBLK)]

    def dst(b):
        return out_hbm.at[pl.ds(base + b * RBLK, RBLK)]

    def compute(in_v, out_v):
        def j_body(j, c):
            col0 = j * (2 * L)
            idx_a = perm_v[pl.ds(col0, L)]
            idx_b = perm_v[pl.ds(col0 + L, L)]
            for r in range(RBLK):
                row_idx = jnp.full((L,), r, jnp.int32)
                out_v[r, pl.ds(col0, L)] = plsc.load_gather(
                    in_v, [row_idx, idx_a])
                out_v[r, pl.ds(col0 + L, L)] = plsc.load_gather(
                    in_v, [row_idx, idx_b])
            return c
        lax.fori_loop(0, DIM // (2 * L), j_body, 0)

    pltpu.async_copy(src(0), in0, sem_in0)

    def pair_body(p, carry):
        b = 2 * p
        pltpu.async_copy(src(b + 1), in1, sem_in1)
        pltpu.make_async_copy(src(b), in0, sem_in0).wait()

        @pl.when(p > 0)
        def _():
            pltpu.make_async_copy(out0, dst(b - 2), sem_out0).wait()

        compute(in0, out0)
        pltpu.async_copy(out0, dst(b), sem_out0)

        @pl.when(p < NB // 2 - 1)
        def _():
            pltpu.async_copy(src(b + 2), in0, sem_in0)

        pltpu.make_async_copy(src(b + 1), in1, sem_in1).wait()

        @pl.when(p > 0)
        def _():
            pltpu.make_async_copy(out1, dst(b - 1), sem_out1).wait()

        compute(in1, out1)
        pltpu.async_copy(out1, dst(b + 1), sem_out1)
        return carry

    lax.fori_loop(0, NB // 2, pair_body, 0)
    pltpu.make_async_copy(out0, dst(NB - 2), sem_out0).wait()
    pltpu.make_async_copy(out1, dst(NB - 1), sem_out1).wait()


def kernel(x, perm):
    out = _permute_cols(x, perm.astype(jnp.int32))
    return out, 0


# speedup vs baseline: 5.6885x; 3.0356x over previous
import functools

import jax
import jax.numpy as jnp
from jax import lax
from jax.experimental import pallas as pl
from jax.experimental.pallas import tpu as pltpu
from jax.experimental.pallas import tpu_sc as plsc

N_ROWS = 8192
DIM = 4096
NC = 2
NS = 16
L = 16
NW = NC * NS
ROWS_PER_W = N_ROWS // NW
RBLK = 4
NB = ROWS_PER_W // RBLK

_mesh = plsc.VectorSubcoreMesh(core_axis_name="c", subcore_axis_name="s")


@functools.partial(
    pl.kernel,
    mesh=_mesh,
    out_type=jax.ShapeDtypeStruct((N_ROWS, DIM), jnp.float32),
    compiler_params=pltpu.CompilerParams(needs_layout_passes=False),
    scratch_types=[
        pltpu.VMEM((DIM,), jnp.int32),
        pltpu.VMEM((RBLK, DIM), jnp.float32),
        pltpu.VMEM((RBLK, DIM), jnp.float32),
        pltpu.VMEM((RBLK, DIM), jnp.float32),
        pltpu.VMEM((RBLK, DIM), jnp.float32),
        pltpu.SemaphoreType.DMA,
        pltpu.SemaphoreType.DMA,
        pltpu.SemaphoreType.DMA,
        pltpu.SemaphoreType.DMA,
    ],
)
def _permute_cols(x_hbm, perm_hbm, out_hbm, perm_v,
                  in0, in1, out0, out1,
                  sem_in0, sem_in1, sem_out0, sem_out1):
    wid = lax.axis_index("s") * NC + lax.axis_index("c")
    base = wid * ROWS_PER_W
    pltpu.sync_copy(perm_hbm, perm_v)

    def src(b):
        return x_hbm.at[pl.ds(base + b * RBLK, RBLK)]

    def dst(b):
        return out_hbm.at[pl.ds(base + b * RBLK, RBLK)]

    def compute(in_v, out_v):
        @plsc.parallel_loop(0, DIM // L, unroll=8)
        def j_body(j):
            col0 = j * L
            idx = perm_v[pl.ds(col0, L)]
            for r in range(RBLK):
                row_idx = jnp.full((L,), r, jnp.int32)
                out_v[r, pl.ds(col0, L)] = plsc.load_gather(
                    in_v, [row_idx, idx])

    pltpu.async_copy(src(0), in0, sem_in0)

    def pair_body(p, carry):
        b = 2 * p
        pltpu.async_copy(src(b + 1), in1, sem_in1)
        pltpu.make_async_copy(src(b), in0, sem_in0).wait()

        @pl.when(p > 0)
        def _():
            pltpu.make_async_copy(out0, dst(b - 2), sem_out0).wait()

        compute(in0, out0)
        pltpu.async_copy(out0, dst(b), sem_out0)

        @pl.when(p < NB // 2 - 1)
        def _():
            pltpu.async_copy(src(b + 2), in0, sem_in0)

        pltpu.make_async_copy(src(b + 1), in1, sem_in1).wait()

        @pl.when(p > 0)
        def _():
            pltpu.make_async_copy(out1, dst(b - 1), sem_out1).wait()

        compute(in1, out1)
        pltpu.async_copy(out1, dst(b + 1), sem_out1)
        return carry

    lax.fori_loop(0, NB // 2, pair_body, 0)
    pltpu.make_async_copy(out0, dst(NB - 2), sem_out0).wait()
    pltpu.make_async_copy(out1, dst(NB - 1), sem_out1).wait()


def kernel(x, perm):
    out = _permute_cols(x, perm.astype(jnp.int32))
    return out, 0
